# Initial kernel scaffold; baseline (speedup 1.0000x reference)
import jax
import jax.numpy as jnp
from jax.experimental import pallas as pl


def kernel(x, edge_index, W1, b1, W2, b2, bern_w, W3, b3):
    raise NotImplementedError("write your pallas kernel here")


# speedup vs baseline: 9.1074x; 9.1074x over previous
import functools
from typing import Any

import jax
import jax.numpy as jnp
from jax import lax
from jax.experimental import pallas as pl
from jax.experimental.pallas import tpu as pltpu
from jax.experimental.pallas import tpu_sc as plsc

N = 100000
D_IN = 128
H = 32
HH = 16
C_OUT = 2

B_N = 800
N_TAB = 102400
N_BLOCKS = N_TAB // B_N
REAL_BLOCKS = N // B_N

E = 1600000
ER = 128
E_ROWS = 12544
E_PAD = E_ROWS * ER
K_CH = 8

NSC = 2
NTILE = 16
ROWS_PER_TILE_AGG = E_ROWS // NTILE
ROWS_PER_TILE_DEG = E_ROWS // (NSC * NTILE)
TPS = N_TAB // NTILE

_mesh = plsc.VectorSubcoreMesh(core_axis_name="c", subcore_axis_name="s")
_sc_params = pltpu.CompilerParams(use_tc_tiling_on_sc=False)


def _deg_body(dst_hbm, zeros1_hbm, ones_hbm, deg_hbm, dacc, didx, ones_v):
    c = lax.axis_index("c")
    s = lax.axis_index("s")
    pltpu.sync_copy(zeros1_hbm, dacc.at[pl.ds(s * TPS, TPS)])
    pltpu.sync_copy(ones_hbm, ones_v)
    plsc.subcore_barrier()

    g = c * NTILE + s

    def step(it, carry):
        base = g * ROWS_PER_TILE_DEG + it * K_CH
        pltpu.sync_copy(dst_hbm.at[pl.ds(base, K_CH)], didx)
        for j in range(K_CH):
            pltpu.sync_copy(ones_v, dacc.at[didx.at[j]], add=True)
        return carry

    lax.fori_loop(0, ROWS_PER_TILE_DEG // K_CH, step, 0)
    plsc.subcore_barrier()
    pltpu.sync_copy(dacc.at[pl.ds(s * TPS, TPS)],
                    deg_hbm.at[c].at[pl.ds(s * TPS, TPS)])


_deg_call = pl.kernel(
    _deg_body,
    out_type=jax.ShapeDtypeStruct((NSC, N_TAB), jnp.float32),
    mesh=_mesh,
    scratch_types=[
        pltpu.VMEM_SHARED((N_TAB,), jnp.float32),
        pltpu.VMEM((K_CH, ER), jnp.int32),
        pltpu.VMEM((ER,), jnp.float32),
    ],
    compiler_params=_sc_params,
)


def _agg_body(m_hbm, src_hbm, dst_hbm, zeros2_hbm, g_hbm, acc, rows, sidx,
              didx, gsem):
    c = lax.axis_index("c")
    s = lax.axis_index("s")
    pltpu.sync_copy(zeros2_hbm, acc.at[pl.ds(s * TPS, TPS)])
    plsc.subcore_barrier()

    def step(it, carry):
        base = s * ROWS_PER_TILE_AGG + it * K_CH
        pltpu.sync_copy(src_hbm.at[pl.ds(base, K_CH)], sidx)
        pltpu.sync_copy(dst_hbm.at[pl.ds(base, K_CH)], didx)
        cps = [pltpu.async_copy(m_hbm.at[c].at[sidx.at[j]], rows.at[j], gsem)
               for j in range(K_CH)]
        for cp in cps:
            cp.wait()
        for j in range(K_CH):
            pltpu.sync_copy(rows.at[j], acc.at[didx.at[j]], add=True)
        return carry

    lax.fori_loop(0, ROWS_PER_TILE_AGG // K_CH, step, 0)
    plsc.subcore_barrier()
    pltpu.sync_copy(acc.at[pl.ds(s * TPS, TPS)],
                    g_hbm.at[c].at[pl.ds(s * TPS, TPS)])


_agg_call = pl.kernel(
    _agg_body,
    out_type=jax.ShapeDtypeStruct((NSC, N_TAB, HH), jnp.float32),
    mesh=_mesh,
    scratch_types=[
        pltpu.VMEM_SHARED((N_TAB, HH), jnp.float32),
        pltpu.VMEM((K_CH, ER, HH), jnp.float32),
        pltpu.VMEM((K_CH, ER), jnp.int32),
        pltpu.VMEM((K_CH, ER), jnp.int32),
        pltpu.SemaphoreType.DMA,
    ],
    compiler_params=_sc_params,
)


def _mlp_body(x_ref, w1_ref, b1_ref, w2_ref, b2_ref, degp_ref,
              hs_ref, m1_ref, dinv_ref):
    i = pl.program_id(0)
    x = x_ref[...]
    h1 = jnp.maximum(
        jnp.dot(x, w1_ref[...], preferred_element_type=jnp.float32)
        + b1_ref[...], 0.0)
    h = jnp.maximum(
        jnp.dot(h1, w2_ref[...], preferred_element_type=jnp.float32)
        + b2_ref[...], 0.0)
    deg = degp_ref[:, 0] + degp_ref[:, 1]
    dinv = lax.rsqrt(jnp.maximum(deg, 1.0))[:, None]
    rows = i * B_N + lax.broadcasted_iota(jnp.int32, (B_N, 1), 0)
    h = jnp.where(rows < N, h, 0.0)
    m = h * dinv
    hs_ref[0] = h[:, :HH]
    hs_ref[1] = h[:, HH:]
    m1_ref[0] = m[:, :HH]
    m1_ref[1] = m[:, HH:]
    dinv_ref[...] = dinv


_mlp_call = pl.pallas_call(
    _mlp_body,
    grid=(N_BLOCKS,),
    in_specs=[
        pl.BlockSpec((B_N, D_IN), lambda i: (jnp.minimum(i, REAL_BLOCKS - 1), 0)),
        pl.BlockSpec((D_IN, H), lambda i: (0, 0)),
        pl.BlockSpec((1, H), lambda i: (0, 0)),
        pl.BlockSpec((H, H), lambda i: (0, 0)),
        pl.BlockSpec((1, H), lambda i: (0, 0)),
        pl.BlockSpec((B_N, NSC), lambda i: (i, 0)),
    ],
    out_specs=[
        pl.BlockSpec((NSC, B_N, HH), lambda i: (0, i, 0)),
        pl.BlockSpec((NSC, B_N, HH), lambda i: (0, i, 0)),
        pl.BlockSpec((B_N, 1), lambda i: (i, 0)),
    ],
    out_shape=[
        jax.ShapeDtypeStruct((NSC, N_TAB, HH), jnp.float32),
        jax.ShapeDtypeStruct((NSC, N_TAB, HH), jnp.float32),
        jax.ShapeDtypeStruct((N_TAB, 1), jnp.float32),
    ],
)


def _comb_body(h_ref, g_ref, dinv_ref, t_ref, m_ref):
    dinv = dinv_ref[...][None]
    t = h_ref[...] + g_ref[...] * dinv
    t_ref[...] = t
    m_ref[...] = t * dinv


_comb_call = pl.pallas_call(
    _comb_body,
    grid=(N_BLOCKS,),
    in_specs=[
        pl.BlockSpec((NSC, B_N, HH), lambda i: (0, i, 0)),
        pl.BlockSpec((NSC, B_N, HH), lambda i: (0, i, 0)),
        pl.BlockSpec((B_N, 1), lambda i: (i, 0)),
    ],
    out_specs=[
        pl.BlockSpec((NSC, B_N, HH), lambda i: (0, i, 0)),
        pl.BlockSpec((NSC, B_N, HH), lambda i: (0, i, 0)),
    ],
    out_shape=[
        jax.ShapeDtypeStruct((NSC, N_TAB, HH), jnp.float32),
        jax.ShapeDtypeStruct((NSC, N_TAB, HH), jnp.float32),
    ],
)


def _fin_body(t2_ref, g3_ref, dinv_ref, bw_ref, w3_ref, b3_ref, o_ref):
    w0 = jnp.maximum(bw_ref[0], 0.0)
    w1 = jnp.maximum(bw_ref[1], 0.0)
    w2 = jnp.maximum(bw_ref[2], 0.0)
    ca = 0.25 * w0
    cb = 0.5 * w1 + 0.25 * w2
    dinv = dinv_ref[...][None]
    t2 = t2_ref[...]
    f = ca * t2 + cb * (t2 - g3_ref[...] * dinv)
    f = jnp.maximum(f, 0.0)
    fcat = jnp.concatenate([f[0], f[1]], axis=1)
    o_ref[...] = (
        jnp.dot(fcat, w3_ref[...], preferred_element_type=jnp.float32)
        + b3_ref[...])


_fin_call = pl.pallas_call(
    _fin_body,
    grid=(REAL_BLOCKS,),
    in_specs=[
        pl.BlockSpec((NSC, B_N, HH), lambda i: (0, i, 0)),
        pl.BlockSpec((NSC, B_N, HH), lambda i: (0, i, 0)),
        pl.BlockSpec((B_N, 1), lambda i: (i, 0)),
        pl.BlockSpec(memory_space=pltpu.SMEM),
        pl.BlockSpec((H, C_OUT), lambda i: (0, 0)),
        pl.BlockSpec((1, C_OUT), lambda i: (0, 0)),
    ],
    out_specs=pl.BlockSpec((B_N, C_OUT), lambda i: (i, 0)),
    out_shape=jax.ShapeDtypeStruct((N, C_OUT), jnp.float32),
)


def kernel(x, edge_index, W1, b1, W2, b2, bern_w, W3, b3):
    src = edge_index[0].astype(jnp.int32)
    dst = edge_index[1].astype(jnp.int32)
    npad = E_PAD - E
    pad_idx = N + (jnp.arange(npad, dtype=jnp.int32) % (N_TAB - N))
    src2d = jnp.concatenate([src, pad_idx]).reshape(E_ROWS, ER)
    dst2d = jnp.concatenate([dst, pad_idx]).reshape(E_ROWS, ER)

    zeros1 = jnp.zeros((TPS,), jnp.float32)
    zeros2 = jnp.zeros((TPS, HH), jnp.float32)
    ones_e = jnp.ones((ER,), jnp.float32)

    degp = _deg_call(dst2d, zeros1, ones_e)
    hs, m1, dinv = _mlp_call(x, W1.T, b1[None], W2.T, b2[None], degp.T)
    g1 = _agg_call(m1, src2d, dst2d, zeros2)
    t1, m2 = _comb_call(hs, g1, dinv)
    g2 = _agg_call(m2, src2d, dst2d, zeros2)
    t2, m3 = _comb_call(t1, g2, dinv)
    g3 = _agg_call(m3, src2d, dst2d, zeros2)
    out = _fin_call(t2, g3, dinv, bern_w, W3.T, b3[None])
    return out


# speedup vs baseline: 11.0625x; 1.2147x over previous
import functools
from typing import Any

import jax
import jax.numpy as jnp
from jax import lax
from jax.experimental import pallas as pl
from jax.experimental.pallas import tpu as pltpu
from jax.experimental.pallas import tpu_sc as plsc

N = 100000
D_IN = 128
H = 32
HH = 16
C_OUT = 2

B_N = 1024
N_TAB = 102400
N_BLOCKS = N_TAB // B_N
LAST_X_BLK = (N - 1) // B_N
PACK = B_N * HH // 128
N_PACK = N_TAB * HH // 128

E = 1600000
ER = 128
E_ROWS = 12544
E_PAD = E_ROWS * ER
K_CH = 8

NSC = 2
NTILE = 16
ROWS_PER_TILE = E_ROWS // NTILE
TPS = N_TAB // NTILE
NHALF_PS = N_TAB // NSC // NTILE

_mesh = plsc.VectorSubcoreMesh(core_axis_name="c", subcore_axis_name="s")
_sc_params = pltpu.CompilerParams(use_tc_tiling_on_sc=False)


def _deg_body(dst_hbm, zeros1_hbm, ones_hbm, dp_hbm, dacc, didx, ones_v):
    c = lax.axis_index("c")
    s = lax.axis_index("s")
    pltpu.sync_copy(zeros1_hbm, dacc.at[pl.ds(s * TPS, TPS)])
    pltpu.sync_copy(ones_hbm, ones_v)
    plsc.subcore_barrier()

    def step(it, carry):
        base = s * ROWS_PER_TILE + it * K_CH
        pltpu.sync_copy(dst_hbm.at[pl.ds(base, K_CH)], didx)
        for j in range(K_CH):
            pltpu.sync_copy(ones_v, dacc.at[didx.at[j]], add=True)
        return carry

    lax.fori_loop(0, ROWS_PER_TILE // K_CH, step, 0)
    plsc.subcore_barrier()

    base_node = c * (N_TAB // NSC) + s * NHALF_PS
    pltpu.sync_copy(dacc.at[pl.ds(base_node, NHALF_PS)],
                    dp_hbm.at[pl.ds(base_node, NHALF_PS)])


_deg_call = pl.kernel(
    _deg_body,
    out_type=jax.ShapeDtypeStruct((N_TAB,), jnp.float32),
    mesh=_mesh,
    scratch_types=[
        pltpu.VMEM_SHARED((N_TAB,), jnp.float32),
        pltpu.VMEM((K_CH, ER), jnp.int32),
        pltpu.VMEM((ER,), jnp.float32),
    ],
    compiler_params=_sc_params,
)


def _agg_body(m_hbm, src_hbm, dst_hbm, zeros2_hbm, g_hbm, acc, rows, sidx,
              didx, gsem):
    c = lax.axis_index("c")
    s = lax.axis_index("s")
    pltpu.sync_copy(zeros2_hbm, acc.at[pl.ds(s * TPS, TPS)])
    plsc.subcore_barrier()

    def step(it, carry):
        base = s * ROWS_PER_TILE + it * K_CH
        pltpu.sync_copy(src_hbm.at[pl.ds(base, K_CH)], sidx)
        pltpu.sync_copy(dst_hbm.at[pl.ds(base, K_CH)], didx)
        cps = [pltpu.async_copy(m_hbm.at[c].at[sidx.at[j]], rows.at[j], gsem)
               for j in range(K_CH)]
        for cp in cps:
            cp.wait()
        for j in range(K_CH):
            pltpu.sync_copy(rows.at[j], acc.at[didx.at[j]], add=True)
        return carry

    lax.fori_loop(0, ROWS_PER_TILE // K_CH, step, 0)
    plsc.subcore_barrier()
    pltpu.sync_copy(acc.at[pl.ds(s * TPS, TPS)],
                    g_hbm.at[c].at[pl.ds(s * TPS, TPS)])


_agg_call = pl.kernel(
    _agg_body,
    out_type=jax.ShapeDtypeStruct((NSC, N_TAB, HH), jnp.float32),
    mesh=_mesh,
    scratch_types=[
        pltpu.VMEM_SHARED((N_TAB, HH), jnp.float32),
        pltpu.VMEM((K_CH, ER, HH), jnp.float32),
        pltpu.VMEM((K_CH, ER), jnp.int32),
        pltpu.VMEM((K_CH, ER), jnp.int32),
        pltpu.SemaphoreType.DMA,
    ],
    compiler_params=_sc_params,
)


def _pack(hh):
    h3 = hh.reshape(PACK, 8, HH)
    return jnp.concatenate(
        [h3[:, j:j + 1, :].reshape(PACK, HH) for j in range(8)], axis=1)


def _unpack(fp):
    parts = [fp[:, HH * j:HH * (j + 1)].reshape(PACK, 1, HH)
             for j in range(8)]
    return jnp.concatenate(parts, axis=1).reshape(B_N, HH)


def _mlp_body(x_ref, w1_ref, b1_ref, w2_ref, b2_ref, dg_ref,
              hs_ref, m1_ref, dp_ref):
    i = pl.program_id(0)
    x = x_ref[...]
    h1 = jnp.maximum(
        jnp.dot(x, w1_ref[...], preferred_element_type=jnp.float32)
        + b1_ref[...], 0.0)
    h = jnp.maximum(
        jnp.dot(h1, w2_ref[...], preferred_element_type=jnp.float32)
        + b2_ref[...], 0.0)
    rows = i * B_N + lax.broadcasted_iota(jnp.int32, (B_N, 1), 0)
    h = jnp.where(rows < N, h, 0.0)
    dpn = lax.rsqrt(jnp.maximum(dg_ref[...], 1.0))
    dpb = _pack(jnp.broadcast_to(dpn, (B_N, HH)))
    dp_ref[...] = dpb
    hp0 = _pack(h[:, :HH])
    hp1 = _pack(h[:, HH:])
    hs_ref[0] = hp0
    hs_ref[1] = hp1
    m1_ref[0] = hp0 * dpb
    m1_ref[1] = hp1 * dpb


_mlp_call = pl.pallas_call(
    _mlp_body,
    grid=(N_BLOCKS,),
    in_specs=[
        pl.BlockSpec((B_N, D_IN), lambda i: (jnp.minimum(i, LAST_X_BLK), 0)),
        pl.BlockSpec((D_IN, H), lambda i: (0, 0)),
        pl.BlockSpec((1, H), lambda i: (0, 0)),
        pl.BlockSpec((H, H), lambda i: (0, 0)),
        pl.BlockSpec((1, H), lambda i: (0, 0)),
        pl.BlockSpec((B_N, 1), lambda i: (i, 0)),
    ],
    out_specs=[
        pl.BlockSpec((NSC, PACK, 128), lambda i: (0, i, 0)),
        pl.BlockSpec((NSC, PACK, 128), lambda i: (0, i, 0)),
        pl.BlockSpec((PACK, 128), lambda i: (i, 0)),
    ],
    out_shape=[
        jax.ShapeDtypeStruct((NSC, N_PACK, 128), jnp.float32),
        jax.ShapeDtypeStruct((NSC, N_PACK, 128), jnp.float32),
        jax.ShapeDtypeStruct((N_PACK, 128), jnp.float32),
    ],
)


def _comb_body(h_ref, g_ref, dp_ref, t_ref, m_ref):
    dp = dp_ref[...][None]
    t = h_ref[...] + g_ref[...] * dp
    t_ref[...] = t
    m_ref[...] = t * dp


_comb_call = pl.pallas_call(
    _comb_body,
    grid=(N_BLOCKS,),
    in_specs=[
        pl.BlockSpec((NSC, PACK, 128), lambda i: (0, i, 0)),
        pl.BlockSpec((NSC, PACK, 128), lambda i: (0, i, 0)),
        pl.BlockSpec((PACK, 128), lambda i: (i, 0)),
    ],
    out_specs=[
        pl.BlockSpec((NSC, PACK, 128), lambda i: (0, i, 0)),
        pl.BlockSpec((NSC, PACK, 128), lambda i: (0, i, 0)),
    ],
    out_shape=[
        jax.ShapeDtypeStruct((NSC, N_PACK, 128), jnp.float32),
        jax.ShapeDtypeStruct((NSC, N_PACK, 128), jnp.float32),
    ],
)


def _fin_body(t2_ref, g3_ref, dp_ref, bw_ref, w3_ref, b3_ref, o_ref):
    w0 = jnp.maximum(bw_ref[0], 0.0)
    w1 = jnp.maximum(bw_ref[1], 0.0)
    w2 = jnp.maximum(bw_ref[2], 0.0)
    ca = 0.25 * w0
    cb = 0.5 * w1 + 0.25 * w2
    dp = dp_ref[...][None]
    t2 = t2_ref[...]
    f = ca * t2 + cb * (t2 - g3_ref[...] * dp)
    f = jnp.maximum(f, 0.0)
    fcat = jnp.concatenate([_unpack(f[0]), _unpack(f[1])], axis=1)
    o_ref[...] = (
        jnp.dot(fcat, w3_ref[...], preferred_element_type=jnp.float32)
        + b3_ref[...])


_fin_call = pl.pallas_call(
    _fin_body,
    grid=(N_BLOCKS,),
    in_specs=[
        pl.BlockSpec((NSC, PACK, 128), lambda i: (0, i, 0)),
        pl.BlockSpec((NSC, PACK, 128), lambda i: (0, i, 0)),
        pl.BlockSpec((PACK, 128), lambda i: (i, 0)),
        pl.BlockSpec(memory_space=pltpu.SMEM),
        pl.BlockSpec((H, C_OUT), lambda i: (0, 0)),
        pl.BlockSpec((1, C_OUT), lambda i: (0, 0)),
    ],
    out_specs=pl.BlockSpec((B_N, C_OUT), lambda i: (i, 0)),
    out_shape=jax.ShapeDtypeStruct((N_TAB, C_OUT), jnp.float32),
)


def _to_sc(a):
    return a.reshape(NSC, N_TAB, HH)


def _to_tc(a):
    return a.reshape(NSC, N_PACK, 128)


def kernel(x, edge_index, W1, b1, W2, b2, bern_w, W3, b3):
    src = edge_index[0].astype(jnp.int32)
    dst = edge_index[1].astype(jnp.int32)
    npad = E_PAD - E
    pad_idx = N + (jnp.arange(npad, dtype=jnp.int32) % (N_TAB - N))
    src2d = jnp.concatenate([src, pad_idx]).reshape(E_ROWS, ER)
    dst2d = jnp.concatenate([dst, pad_idx]).reshape(E_ROWS, ER)

    zeros1 = jnp.zeros((TPS,), jnp.float32)
    zeros2 = jnp.zeros((TPS, HH), jnp.float32)
    ones_e = jnp.ones((ER,), jnp.float32)

    dg = _deg_call(dst2d, zeros1, ones_e).reshape(N_TAB, 1)
    hs, m1, dp = _mlp_call(x, W1.T, b1[None], W2.T, b2[None], dg)
    g1 = _agg_call(_to_sc(m1), src2d, dst2d, zeros2)
    t1, m2 = _comb_call(hs, _to_tc(g1), dp)
    g2 = _agg_call(_to_sc(m2), src2d, dst2d, zeros2)
    t2, m3 = _comb_call(t1, _to_tc(g2), dp)
    g3 = _agg_call(_to_sc(m3), src2d, dst2d, zeros2)
    out = _fin_call(t2, _to_tc(g3), dp, bern_w, W3.T, b3[None])
    return out[:N]


# speedup vs baseline: 12.1121x; 1.0949x over previous
import functools
from typing import Any

import jax
import jax.numpy as jnp
from jax import lax
from jax.experimental import pallas as pl
from jax.experimental.pallas import tpu as pltpu
from jax.experimental.pallas import tpu_sc as plsc

N = 100000
D_IN = 128
H = 32
HH = 16
C_OUT = 2

B_N = 1024
N_TAB = 102400
N_BLOCKS = N_TAB // B_N
LAST_X_BLK = (N - 1) // B_N
PACK = B_N * HH // 128
N_PACK = N_TAB * HH // 128

E = 1600000
ER = 128
E_ROWS = 12544
E_PAD = E_ROWS * ER
K_CH = 8

NSC = 2
NTILE = 16
ROWS_PER_TILE = E_ROWS // NTILE
TPS = N_TAB // NTILE
NHALF_PS = N_TAB // NSC // NTILE

_mesh = plsc.VectorSubcoreMesh(core_axis_name="c", subcore_axis_name="s")
_sc_params = pltpu.CompilerParams(use_tc_tiling_on_sc=False)


def _deg_body(dst_hbm, zeros1_hbm, ones_hbm, dp_hbm, dacc, didx, ones_v,
              dsem):
    c = lax.axis_index("c")
    s = lax.axis_index("s")
    pltpu.sync_copy(zeros1_hbm, dacc.at[pl.ds(s * TPS, TPS)])
    pltpu.sync_copy(ones_hbm, ones_v)
    plsc.subcore_barrier()

    def step(it, carry):
        base = s * ROWS_PER_TILE + it * K_CH
        pltpu.sync_copy(dst_hbm.at[pl.ds(base, K_CH)], didx)
        cps = [pltpu.async_copy(ones_v, dacc.at[didx.at[j]], dsem, add=True)
               for j in range(K_CH)]
        for cp in cps:
            cp.wait()
        return carry

    lax.fori_loop(0, ROWS_PER_TILE // K_CH, step, 0)
    plsc.subcore_barrier()

    base_node = c * (N_TAB // NSC) + s * NHALF_PS
    pltpu.sync_copy(dacc.at[pl.ds(base_node, NHALF_PS)],
                    dp_hbm.at[pl.ds(base_node, NHALF_PS)])


_deg_call = pl.kernel(
    _deg_body,
    out_type=jax.ShapeDtypeStruct((N_TAB,), jnp.float32),
    mesh=_mesh,
    scratch_types=[
        pltpu.VMEM_SHARED((N_TAB,), jnp.float32),
        pltpu.VMEM((K_CH, ER), jnp.int32),
        pltpu.VMEM((ER,), jnp.float32),
        pltpu.SemaphoreType.DMA,
    ],
    compiler_params=_sc_params,
)


def _agg_body(m_hbm, src_hbm, dst_hbm, zeros2_hbm, g_hbm, acc, rows, sidx,
              didx, gsem, ssem):
    c = lax.axis_index("c")
    s = lax.axis_index("s")
    pltpu.sync_copy(zeros2_hbm, acc.at[pl.ds(s * TPS, TPS)])
    plsc.subcore_barrier()

    def step(it, carry):
        base = s * ROWS_PER_TILE + it * K_CH
        pltpu.sync_copy(src_hbm.at[pl.ds(base, K_CH)], sidx)
        pltpu.sync_copy(dst_hbm.at[pl.ds(base, K_CH)], didx)
        gcps = [pltpu.async_copy(m_hbm.at[c].at[sidx.at[j]], rows.at[j], gsem)
                for j in range(K_CH)]
        for cp in gcps:
            cp.wait()
        scps = [pltpu.async_copy(rows.at[j], acc.at[didx.at[j]], ssem,
                                 add=True)
                for j in range(K_CH)]
        for cp in scps:
            cp.wait()
        return carry

    lax.fori_loop(0, ROWS_PER_TILE // K_CH, step, 0)
    plsc.subcore_barrier()
    pltpu.sync_copy(acc.at[pl.ds(s * TPS, TPS)],
                    g_hbm.at[c].at[pl.ds(s * TPS, TPS)])


_agg_call = pl.kernel(
    _agg_body,
    out_type=jax.ShapeDtypeStruct((NSC, N_TAB, HH), jnp.float32),
    mesh=_mesh,
    scratch_types=[
        pltpu.VMEM_SHARED((N_TAB, HH), jnp.float32),
        pltpu.VMEM((K_CH, ER, HH), jnp.float32),
        pltpu.VMEM((K_CH, ER), jnp.int32),
        pltpu.VMEM((K_CH, ER), jnp.int32),
        pltpu.SemaphoreType.DMA,
        pltpu.SemaphoreType.DMA,
    ],
    compiler_params=_sc_params,
)


def _pack(hh):
    h3 = hh.reshape(PACK, 8, HH)
    return jnp.concatenate(
        [h3[:, j:j + 1, :].reshape(PACK, HH) for j in range(8)], axis=1)


def _unpack(fp):
    parts = [fp[:, HH * j:HH * (j + 1)].reshape(PACK, 1, HH)
             for j in range(8)]
    return jnp.concatenate(parts, axis=1).reshape(B_N, HH)


def _mlp_body(x_ref, w1_ref, b1_ref, w2_ref, b2_ref, dg_ref,
              hs_ref, m1_ref, dp_ref):
    i = pl.program_id(0)
    x = x_ref[...]
    h1 = jnp.maximum(
        jnp.dot(x, w1_ref[...], preferred_element_type=jnp.float32)
        + b1_ref[...], 0.0)
    h = jnp.maximum(
        jnp.dot(h1, w2_ref[...], preferred_element_type=jnp.float32)
        + b2_ref[...], 0.0)
    rows = i * B_N + lax.broadcasted_iota(jnp.int32, (B_N, 1), 0)
    h = jnp.where(rows < N, h, 0.0)
    dpn = lax.rsqrt(jnp.maximum(dg_ref[...], 1.0))
    dpb = _pack(jnp.broadcast_to(dpn, (B_N, HH)))
    dp_ref[...] = dpb
    hp0 = _pack(h[:, :HH])
    hp1 = _pack(h[:, HH:])
    hs_ref[0] = hp0
    hs_ref[1] = hp1
    m1_ref[0] = hp0 * dpb
    m1_ref[1] = hp1 * dpb


_mlp_call = pl.pallas_call(
    _mlp_body,
    grid=(N_BLOCKS,),
    in_specs=[
        pl.BlockSpec((B_N, D_IN), lambda i: (jnp.minimum(i, LAST_X_BLK), 0)),
        pl.BlockSpec((D_IN, H), lambda i: (0, 0)),
        pl.BlockSpec((1, H), lambda i: (0, 0)),
        pl.BlockSpec((H, H), lambda i: (0, 0)),
        pl.BlockSpec((1, H), lambda i: (0, 0)),
        pl.BlockSpec((B_N, 1), lambda i: (i, 0)),
    ],
    out_specs=[
        pl.BlockSpec((NSC, PACK, 128), lambda i: (0, i, 0)),
        pl.BlockSpec((NSC, PACK, 128), lambda i: (0, i, 0)),
        pl.BlockSpec((PACK, 128), lambda i: (i, 0)),
    ],
    out_shape=[
        jax.ShapeDtypeStruct((NSC, N_PACK, 128), jnp.float32),
        jax.ShapeDtypeStruct((NSC, N_PACK, 128), jnp.float32),
        jax.ShapeDtypeStruct((N_PACK, 128), jnp.float32),
    ],
)


def _comb_body(h_ref, g_ref, dp_ref, t_ref, m_ref):
    dp = dp_ref[...][None]
    t = h_ref[...] + g_ref[...] * dp
    t_ref[...] = t
    m_ref[...] = t * dp


_comb_call = pl.pallas_call(
    _comb_body,
    grid=(N_BLOCKS,),
    in_specs=[
        pl.BlockSpec((NSC, PACK, 128), lambda i: (0, i, 0)),
        pl.BlockSpec((NSC, PACK, 128), lambda i: (0, i, 0)),
        pl.BlockSpec((PACK, 128), lambda i: (i, 0)),
    ],
    out_specs=[
        pl.BlockSpec((NSC, PACK, 128), lambda i: (0, i, 0)),
        pl.BlockSpec((NSC, PACK, 128), lambda i: (0, i, 0)),
    ],
    out_shape=[
        jax.ShapeDtypeStruct((NSC, N_PACK, 128), jnp.float32),
        jax.ShapeDtypeStruct((NSC, N_PACK, 128), jnp.float32),
    ],
)


def _fin_body(t2_ref, g3_ref, dp_ref, bw_ref, w3_ref, b3_ref, o_ref):
    w0 = jnp.maximum(bw_ref[0], 0.0)
    w1 = jnp.maximum(bw_ref[1], 0.0)
    w2 = jnp.maximum(bw_ref[2], 0.0)
    ca = 0.25 * w0
    cb = 0.5 * w1 + 0.25 * w2
    dp = dp_ref[...][None]
    t2 = t2_ref[...]
    f = ca * t2 + cb * (t2 - g3_ref[...] * dp)
    f = jnp.maximum(f, 0.0)
    fcat = jnp.concatenate([_unpack(f[0]), _unpack(f[1])], axis=1)
    o_ref[...] = (
        jnp.dot(fcat, w3_ref[...], preferred_element_type=jnp.float32)
        + b3_ref[...])


_fin_call = pl.pallas_call(
    _fin_body,
    grid=(N_BLOCKS,),
    in_specs=[
        pl.BlockSpec((NSC, PACK, 128), lambda i: (0, i, 0)),
        pl.BlockSpec((NSC, PACK, 128), lambda i: (0, i, 0)),
        pl.BlockSpec((PACK, 128), lambda i: (i, 0)),
        pl.BlockSpec(memory_space=pltpu.SMEM),
        pl.BlockSpec((H, C_OUT), lambda i: (0, 0)),
        pl.BlockSpec((1, C_OUT), lambda i: (0, 0)),
    ],
    out_specs=pl.BlockSpec((B_N, C_OUT), lambda i: (i, 0)),
    out_shape=jax.ShapeDtypeStruct((N_TAB, C_OUT), jnp.float32),
)


def _to_sc(a):
    return a.reshape(NSC, N_TAB, HH)


def _to_tc(a):
    return a.reshape(NSC, N_PACK, 128)


def kernel(x, edge_index, W1, b1, W2, b2, bern_w, W3, b3):
    src = edge_index[0].astype(jnp.int32)
    dst = edge_index[1].astype(jnp.int32)
    npad = E_PAD - E
    pad_idx = N + (jnp.arange(npad, dtype=jnp.int32) % (N_TAB - N))
    src2d = jnp.concatenate([src, pad_idx]).reshape(E_ROWS, ER)
    dst2d = jnp.concatenate([dst, pad_idx]).reshape(E_ROWS, ER)

    zeros1 = jnp.zeros((TPS,), jnp.float32)
    zeros2 = jnp.zeros((TPS, HH), jnp.float32)
    ones_e = jnp.ones((ER,), jnp.float32)

    dg = _deg_call(dst2d, zeros1, ones_e).reshape(N_TAB, 1)
    hs, m1, dp = _mlp_call(x, W1.T, b1[None], W2.T, b2[None], dg)
    g1 = _agg_call(_to_sc(m1), src2d, dst2d, zeros2)
    t1, m2 = _comb_call(hs, _to_tc(g1), dp)
    g2 = _agg_call(_to_sc(m2), src2d, dst2d, zeros2)
    t2, m3 = _comb_call(t1, _to_tc(g2), dp)
    g3 = _agg_call(_to_sc(m3), src2d, dst2d, zeros2)
    out = _fin_call(t2, _to_tc(g3), dp, bern_w, W3.T, b3[None])
    return out[:N]


# speedup vs baseline: 13.1271x; 1.0838x over previous
import functools
from typing import Any

import jax
import jax.numpy as jnp
from jax import lax
from jax.experimental import pallas as pl
from jax.experimental.pallas import tpu as pltpu
from jax.experimental.pallas import tpu_sc as plsc

N = 100000
D_IN = 128
H = 32
HH = 16
C_OUT = 2

B_N = 1024
N_TAB = 102400
N_BLOCKS = N_TAB // B_N
LAST_X_BLK = (N - 1) // B_N
PACK = B_N * HH // 128
N_PACK = N_TAB * HH // 128

E = 1600000
ER = 128
E_ROWS = 12544
E_PAD = E_ROWS * ER
K_CH = 8
K_AG = 4

NSC = 2
NTILE = 16
ROWS_PER_TILE = E_ROWS // NTILE
HALF_RPT = ROWS_PER_TILE // NSC
TPS = N_TAB // NTILE
NHALF_PS = N_TAB // NSC // NTILE

_mesh = plsc.VectorSubcoreMesh(core_axis_name="c", subcore_axis_name="s")
_sc_params = pltpu.CompilerParams(use_tc_tiling_on_sc=False)


def _deg_body(dst_hbm, zeros1_hbm, ones_hbm, dp_hbm, dacc, didx, ones_v,
              dsem):
    c = lax.axis_index("c")
    s = lax.axis_index("s")
    pltpu.sync_copy(zeros1_hbm, dacc.at[pl.ds(s * TPS, TPS)])
    pltpu.sync_copy(ones_hbm, ones_v)
    plsc.subcore_barrier()

    def step(it, carry):
        base = (c * NTILE + s) * HALF_RPT + it * K_CH
        pltpu.sync_copy(dst_hbm.at[pl.ds(base, K_CH)], didx)
        cps = [pltpu.async_copy(ones_v, dacc.at[didx.at[j]], dsem, add=True)
               for j in range(K_CH)]
        for cp in cps:
            cp.wait()
        return carry

    lax.fori_loop(0, HALF_RPT // K_CH, step, 0)
    plsc.subcore_barrier()

    pltpu.sync_copy(dacc.at[pl.ds(s * TPS, TPS)],
                    dp_hbm.at[c].at[pl.ds(s * TPS, TPS)])


_deg_call = pl.kernel(
    _deg_body,
    out_type=jax.ShapeDtypeStruct((NSC, N_TAB), jnp.float32),
    mesh=_mesh,
    scratch_types=[
        pltpu.VMEM_SHARED((N_TAB,), jnp.float32),
        pltpu.VMEM((K_CH, ER), jnp.int32),
        pltpu.VMEM((ER,), jnp.float32),
        pltpu.SemaphoreType.DMA,
    ],
    compiler_params=_sc_params,
)


def _agg_body(m_hbm, src_hbm, dst_hbm, zeros2_hbm, g_hbm, acc,
              rows0, rows1, sidx0, sidx1, didx0, didx1,
              gsem0, gsem1, ssem0, ssem1):
    c = lax.axis_index("c")
    s = lax.axis_index("s")
    pltpu.sync_copy(zeros2_hbm, acc.at[pl.ds(s * TPS, TPS)])
    plsc.subcore_barrier()

    base0 = s * ROWS_PER_TILE
    pltpu.sync_copy(src_hbm.at[pl.ds(base0, K_AG)], sidx0)
    pltpu.sync_copy(dst_hbm.at[pl.ds(base0, K_AG)], didx0)
    for j in range(K_AG):
        pltpu.async_copy(m_hbm.at[c].at[sidx0.at[j]], rows0.at[j], gsem0)

    def dstep(g, carry):
        base1 = s * ROWS_PER_TILE + (2 * g + 1) * K_AG
        pltpu.sync_copy(src_hbm.at[pl.ds(base1, K_AG)], sidx1)
        pltpu.sync_copy(dst_hbm.at[pl.ds(base1, K_AG)], didx1)
        g1 = [pltpu.async_copy(m_hbm.at[c].at[sidx1.at[j]], rows1.at[j],
                               gsem1)
              for j in range(K_AG)]
        for j in range(K_AG):
            pltpu.make_async_copy(m_hbm.at[c].at[sidx0.at[j]], rows0.at[j],
                                  gsem0).wait()
        s0 = [pltpu.async_copy(rows0.at[j], acc.at[didx0.at[j]], ssem0,
                               add=True)
              for j in range(K_AG)]
        for cp in s0:
            cp.wait()
        base2 = jnp.minimum(s * ROWS_PER_TILE + (2 * g + 2) * K_AG,
                            E_ROWS - K_AG)
        pltpu.sync_copy(src_hbm.at[pl.ds(base2, K_AG)], sidx0)
        pltpu.sync_copy(dst_hbm.at[pl.ds(base2, K_AG)], didx0)
        for j in range(K_AG):
            pltpu.async_copy(m_hbm.at[c].at[sidx0.at[j]], rows0.at[j], gsem0)
        for cp in g1:
            cp.wait()
        s1 = [pltpu.async_copy(rows1.at[j], acc.at[didx1.at[j]], ssem1,
                               add=True)
              for j in range(K_AG)]
        for cp in s1:
            cp.wait()
        return carry

    lax.fori_loop(0, ROWS_PER_TILE // K_AG // 2, dstep, 0)
    for j in range(K_AG):
        pltpu.make_async_copy(m_hbm.at[c].at[sidx0.at[j]], rows0.at[j],
                              gsem0).wait()
    plsc.subcore_barrier()
    pltpu.sync_copy(acc.at[pl.ds(s * TPS, TPS)],
                    g_hbm.at[c].at[pl.ds(s * TPS, TPS)])


_agg_call = pl.kernel(
    _agg_body,
    out_type=jax.ShapeDtypeStruct((NSC, N_TAB, HH), jnp.float32),
    mesh=_mesh,
    scratch_types=[
        pltpu.VMEM_SHARED((N_TAB, HH), jnp.float32),
        pltpu.VMEM((K_AG, ER, HH), jnp.float32),
        pltpu.VMEM((K_AG, ER, HH), jnp.float32),
        pltpu.VMEM((K_AG, ER), jnp.int32),
        pltpu.VMEM((K_AG, ER), jnp.int32),
        pltpu.VMEM((K_AG, ER), jnp.int32),
        pltpu.VMEM((K_AG, ER), jnp.int32),
        pltpu.SemaphoreType.DMA,
        pltpu.SemaphoreType.DMA,
        pltpu.SemaphoreType.DMA,
        pltpu.SemaphoreType.DMA,
    ],
    compiler_params=_sc_params,
)


def _pack(hh):
    h3 = hh.reshape(PACK, 8, HH)
    return jnp.concatenate(
        [h3[:, j:j + 1, :].reshape(PACK, HH) for j in range(8)], axis=1)


def _unpack(fp):
    parts = [fp[:, HH * j:HH * (j + 1)].reshape(PACK, 1, HH)
             for j in range(8)]
    return jnp.concatenate(parts, axis=1).reshape(B_N, HH)


def _mlp_body(x_ref, w1_ref, b1_ref, w2_ref, b2_ref, dg_ref,
              hs_ref, m1_ref, dp_ref):
    i = pl.program_id(0)
    x = x_ref[...]
    h1 = jnp.maximum(
        jnp.dot(x, w1_ref[...], preferred_element_type=jnp.float32)
        + b1_ref[...], 0.0)
    h = jnp.maximum(
        jnp.dot(h1, w2_ref[...], preferred_element_type=jnp.float32)
        + b2_ref[...], 0.0)
    rows = i * B_N + lax.broadcasted_iota(jnp.int32, (B_N, 1), 0)
    h = jnp.where(rows < N, h, 0.0)
    dg = dg_ref[0] + dg_ref[1]
    dpn = lax.rsqrt(jnp.maximum(dg, 1.0))
    dpb = _pack(jnp.broadcast_to(dpn, (B_N, HH)))
    dp_ref[...] = dpb
    hp0 = _pack(h[:, :HH])
    hp1 = _pack(h[:, HH:])
    hs_ref[0] = hp0
    hs_ref[1] = hp1
    m1_ref[0] = hp0 * dpb
    m1_ref[1] = hp1 * dpb


_mlp_call = pl.pallas_call(
    _mlp_body,
    grid=(N_BLOCKS,),
    in_specs=[
        pl.BlockSpec((B_N, D_IN), lambda i: (jnp.minimum(i, LAST_X_BLK), 0)),
        pl.BlockSpec((D_IN, H), lambda i: (0, 0)),
        pl.BlockSpec((1, H), lambda i: (0, 0)),
        pl.BlockSpec((H, H), lambda i: (0, 0)),
        pl.BlockSpec((1, H), lambda i: (0, 0)),
        pl.BlockSpec((NSC, B_N, 1), lambda i: (0, i, 0)),
    ],
    out_specs=[
        pl.BlockSpec((NSC, PACK, 128), lambda i: (0, i, 0)),
        pl.BlockSpec((NSC, PACK, 128), lambda i: (0, i, 0)),
        pl.BlockSpec((PACK, 128), lambda i: (i, 0)),
    ],
    out_shape=[
        jax.ShapeDtypeStruct((NSC, N_PACK, 128), jnp.float32),
        jax.ShapeDtypeStruct((NSC, N_PACK, 128), jnp.float32),
        jax.ShapeDtypeStruct((N_PACK, 128), jnp.float32),
    ],
)


def _comb_body(h_ref, g_ref, dp_ref, t_ref, m_ref):
    dp = dp_ref[...][None]
    t = h_ref[...] + g_ref[...] * dp
    t_ref[...] = t
    m_ref[...] = t * dp


_comb_call = pl.pallas_call(
    _comb_body,
    grid=(N_BLOCKS,),
    in_specs=[
        pl.BlockSpec((NSC, PACK, 128), lambda i: (0, i, 0)),
        pl.BlockSpec((NSC, PACK, 128), lambda i: (0, i, 0)),
        pl.BlockSpec((PACK, 128), lambda i: (i, 0)),
    ],
    out_specs=[
        pl.BlockSpec((NSC, PACK, 128), lambda i: (0, i, 0)),
        pl.BlockSpec((NSC, PACK, 128), lambda i: (0, i, 0)),
    ],
    out_shape=[
        jax.ShapeDtypeStruct((NSC, N_PACK, 128), jnp.float32),
        jax.ShapeDtypeStruct((NSC, N_PACK, 128), jnp.float32),
    ],
)


def _fin_body(t2_ref, g3_ref, dp_ref, bw_ref, w3_ref, b3_ref, o_ref):
    w0 = jnp.maximum(bw_ref[0], 0.0)
    w1 = jnp.maximum(bw_ref[1], 0.0)
    w2 = jnp.maximum(bw_ref[2], 0.0)
    ca = 0.25 * w0
    cb = 0.5 * w1 + 0.25 * w2
    dp = dp_ref[...][None]
    t2 = t2_ref[...]
    f = ca * t2 + cb * (t2 - g3_ref[...] * dp)
    f = jnp.maximum(f, 0.0)
    fcat = jnp.concatenate([_unpack(f[0]), _unpack(f[1])], axis=1)
    o_ref[...] = (
        jnp.dot(fcat, w3_ref[...], preferred_element_type=jnp.float32)
        + b3_ref[...])


_fin_call = pl.pallas_call(
    _fin_body,
    grid=(N_BLOCKS,),
    in_specs=[
        pl.BlockSpec((NSC, PACK, 128), lambda i: (0, i, 0)),
        pl.BlockSpec((NSC, PACK, 128), lambda i: (0, i, 0)),
        pl.BlockSpec((PACK, 128), lambda i: (i, 0)),
        pl.BlockSpec(memory_space=pltpu.SMEM),
        pl.BlockSpec((H, C_OUT), lambda i: (0, 0)),
        pl.BlockSpec((1, C_OUT), lambda i: (0, 0)),
    ],
    out_specs=pl.BlockSpec((B_N, C_OUT), lambda i: (i, 0)),
    out_shape=jax.ShapeDtypeStruct((N_TAB, C_OUT), jnp.float32),
)


def _to_sc(a):
    return a.reshape(NSC, N_TAB, HH)


def _to_tc(a):
    return a.reshape(NSC, N_PACK, 128)


def kernel(x, edge_index, W1, b1, W2, b2, bern_w, W3, b3):
    src = edge_index[0].astype(jnp.int32)
    dst = edge_index[1].astype(jnp.int32)
    npad = E_PAD - E
    pad_idx = N + (jnp.arange(npad, dtype=jnp.int32) % (N_TAB - N))
    src2d = jnp.concatenate([src, pad_idx]).reshape(E_ROWS, ER)
    dst2d = jnp.concatenate([dst, pad_idx]).reshape(E_ROWS, ER)

    zeros1 = jnp.zeros((TPS,), jnp.float32)
    zeros2 = jnp.zeros((TPS, HH), jnp.float32)
    ones_e = jnp.ones((ER,), jnp.float32)

    dg = _deg_call(dst2d, zeros1, ones_e).reshape(NSC, N_TAB, 1)
    hs, m1, dp = _mlp_call(x, W1.T, b1[None], W2.T, b2[None], dg)
    g1 = _agg_call(_to_sc(m1), src2d, dst2d, zeros2)
    t1, m2 = _comb_call(hs, _to_tc(g1), dp)
    g2 = _agg_call(_to_sc(m2), src2d, dst2d, zeros2)
    t2, m3 = _comb_call(t1, _to_tc(g2), dp)
    g3 = _agg_call(_to_sc(m3), src2d, dst2d, zeros2)
    out = _fin_call(t2, _to_tc(g3), dp, bern_w, W3.T, b3[None])
    return out[:N]
